# Initial kernel scaffold; baseline (speedup 1.0000x reference)
import jax
import jax.numpy as jnp
from jax.experimental import pallas as pl


def kernel(x, edge_index, batch, W_rel1, W_root1, b1, W_rel2, W_root2, b2):
    raise NotImplementedError("write your pallas kernel here")


# speedup vs baseline: 4.6240x; 4.6240x over previous
import functools

import jax
import jax.numpy as jnp
from jax import lax
from jax.experimental import pallas as pl
from jax.experimental.pallas import tpu as pltpu
from jax.experimental.pallas import tpu_sc as plsc

N_NODES = 10000
N_EDGES = 320000
D = 128
N_POOL = 8

NC = 2
NS = 16
NW = NC * NS
EPT = N_EDGES // NW
CHUNK = 80
NCHUNKS = EPT // CHUNK
RPT = 624
RTAIL = N_NODES - NS * RPT

_mesh = plsc.VectorSubcoreMesh(
    core_axis_name="c", subcore_axis_name="s", num_cores=NC, num_subcores=NS
)


@functools.partial(
    pl.kernel,
    out_type=jax.ShapeDtypeStruct((NC * N_NODES, D), jnp.float32),
    mesh=_mesh,
    scratch_types=[
        pltpu.VMEM((CHUNK,), jnp.int32),
        pltpu.VMEM((CHUNK,), jnp.int32),
        pltpu.VMEM((CHUNK, D), jnp.float32),
        pltpu.VMEM_SHARED((N_NODES, D), jnp.float32),
        pltpu.SemaphoreType.DMA,
    ],
)
def _sc_aggregate(x_hbm, src_hbm, dst_hbm, z_hbm, out_hbm,
                  src_v, dst_v, rows_v, acc, sem):
    c = lax.axis_index("c")
    s = lax.axis_index("s")
    wid = c * NS + s
    pltpu.sync_copy(z_hbm.at[pl.ds(0, RPT)], acc.at[pl.ds(s * RPT, RPT)])

    @pl.when(s == 0)
    def _():
        pltpu.sync_copy(z_hbm.at[pl.ds(0, RTAIL)],
                        acc.at[pl.ds(NS * RPT, RTAIL)])

    plsc.subcore_barrier()

    def step(j, carry):
        base = pl.multiple_of(wid * EPT + j * CHUNK, 8)
        pltpu.sync_copy(src_hbm.at[pl.ds(base, CHUNK)], src_v)
        pltpu.sync_copy(dst_hbm.at[pl.ds(base, CHUNK)], dst_v)
        pltpu.async_copy(x_hbm.at[src_v], rows_v, sem).wait()
        pltpu.sync_copy(rows_v, acc.at[dst_v], add=True)
        return carry

    lax.fori_loop(0, NCHUNKS, step, 0)
    plsc.subcore_barrier()
    row0 = pl.multiple_of(c * N_NODES + s * RPT, 8)
    pltpu.sync_copy(acc.at[pl.ds(s * RPT, RPT)], out_hbm.at[pl.ds(row0, RPT)])

    @pl.when(s == 0)
    def _():
        r0 = pl.multiple_of(c * N_NODES + NS * RPT, 8)
        pltpu.sync_copy(acc.at[pl.ds(NS * RPT, RTAIL)],
                        out_hbm.at[pl.ds(r0, RTAIL)])


BLK = 1000
GRID = N_NODES // BLK


def _dense_relu_body(p0, p1, x, wrel, wroot, b, out):
    acc = jnp.dot(p0[...] + p1[...], wrel[...],
                  preferred_element_type=jnp.float32)
    acc = acc + jnp.dot(x[...], wroot[...],
                        preferred_element_type=jnp.float32)
    acc = acc + b[...]
    out[...] = jnp.maximum(acc, 0.0)


_dense_relu = pl.pallas_call(
    _dense_relu_body,
    grid=(GRID,),
    in_specs=[
        pl.BlockSpec((BLK, D), lambda i: (i, 0)),
        pl.BlockSpec((BLK, D), lambda i: (i, 0)),
        pl.BlockSpec((BLK, D), lambda i: (i, 0)),
        pl.BlockSpec((D, D), lambda i: (0, 0)),
        pl.BlockSpec((D, D), lambda i: (0, 0)),
        pl.BlockSpec((1, D), lambda i: (0, 0)),
    ],
    out_specs=pl.BlockSpec((BLK, D), lambda i: (i, 0)),
    out_shape=jax.ShapeDtypeStruct((N_NODES, D), jnp.float32),
)


def _dense_pool_body(q0, q1, h, wrel, wroot, b, bat, pooled):
    i = pl.program_id(0)
    out = jnp.dot(q0[...] + q1[...], wrel[...],
                  preferred_element_type=jnp.float32)
    out = out + jnp.dot(h[...], wroot[...],
                        preferred_element_type=jnp.float32)
    out = out + b[...]

    @pl.when(i == 0)
    def _():
        pooled[...] = jnp.full((N_POOL, D), -jnp.inf, jnp.float32)

    bt = bat[...]
    rows = [jnp.max(jnp.where(bt == g, out, -jnp.inf), axis=0)
            for g in range(N_POOL)]
    pooled[...] = jnp.maximum(pooled[...], jnp.stack(rows, axis=0))


_dense_pool = pl.pallas_call(
    _dense_pool_body,
    grid=(GRID,),
    in_specs=[
        pl.BlockSpec((BLK, D), lambda i: (i, 0)),
        pl.BlockSpec((BLK, D), lambda i: (i, 0)),
        pl.BlockSpec((BLK, D), lambda i: (i, 0)),
        pl.BlockSpec((D, D), lambda i: (0, 0)),
        pl.BlockSpec((D, D), lambda i: (0, 0)),
        pl.BlockSpec((1, D), lambda i: (0, 0)),
        pl.BlockSpec((BLK, 1), lambda i: (i, 0)),
    ],
    out_specs=pl.BlockSpec((N_POOL, D), lambda i: (0, 0)),
    out_shape=jax.ShapeDtypeStruct((N_POOL, D), jnp.float32),
)


def kernel(x, edge_index, batch, W_rel1, W_root1, b1, W_rel2, W_root2, b2):
    x = x.astype(jnp.float32)
    src = edge_index[0].astype(jnp.int32)
    dst = edge_index[1].astype(jnp.int32)
    bat = batch.astype(jnp.int32).reshape(N_NODES, 1)
    zrows = jnp.zeros((RPT, D), jnp.float32)
    b1r = b1.reshape(1, D)
    b2r = b2.reshape(1, D)

    agg1 = _sc_aggregate(x, src, dst, zrows)
    h = _dense_relu(agg1[:N_NODES], agg1[N_NODES:], x, W_rel1, W_root1, b1r)
    agg2 = _sc_aggregate(h, src, dst, zrows)
    pooled = _dense_pool(agg2[:N_NODES], agg2[N_NODES:], h,
                         W_rel2, W_root2, b2r, bat)
    return pooled


# speedup vs baseline: 10.0782x; 2.1795x over previous
import functools

import jax
import jax.numpy as jnp
from jax import lax
from jax.experimental import pallas as pl
from jax.experimental.pallas import tpu as pltpu
from jax.experimental.pallas import tpu_sc as plsc

N_NODES = 10000
N_EDGES = 320000
D = 128
N_POOL = 8

NC = 2
NS = 16
NW = NC * NS
EPT = N_EDGES // NW
CHUNK = 80
NCHUNKS = EPT // CHUNK
RPT = 624
RTAIL = N_NODES - NS * RPT

_mesh = plsc.VectorSubcoreMesh(
    core_axis_name="c", subcore_axis_name="s", num_cores=NC, num_subcores=NS
)


@functools.partial(
    pl.kernel,
    out_type=jax.ShapeDtypeStruct((NC * N_NODES, D), jnp.float32),
    mesh=_mesh,
    scratch_types=[
        pltpu.VMEM((EPT,), jnp.int32),
        pltpu.VMEM((NCHUNKS, CHUNK), jnp.int32),
        pltpu.VMEM((2, CHUNK, D), jnp.float32),
        pltpu.VMEM_SHARED((N_NODES, D), jnp.float32),
        pltpu.SemaphoreType.DMA,
        pltpu.SemaphoreType.DMA,
    ],
)
def _sc_aggregate(x_hbm, src_hbm, dst_hbm, z_hbm, out_hbm,
                  srci, dsti, rows, acc, sem0, sem1):
    c = lax.axis_index("c")
    s = lax.axis_index("s")
    wid = c * NS + s
    pltpu.sync_copy(src_hbm.at[pl.ds(pl.multiple_of(wid * EPT, 8), EPT)], srci)
    pltpu.sync_copy(dst_hbm.at[wid], dsti)
    pltpu.sync_copy(z_hbm.at[pl.ds(0, RPT)], acc.at[pl.ds(s * RPT, RPT)])

    @pl.when(s == 0)
    def _():
        pltpu.sync_copy(z_hbm.at[pl.ds(0, RTAIL)],
                        acc.at[pl.ds(NS * RPT, RTAIL)])

    plsc.subcore_barrier()

    def sidx(j):
        return srci.at[pl.ds(pl.multiple_of(j * CHUNK, 8), CHUNK)]

    pltpu.async_copy(x_hbm.at[sidx(0)], rows.at[0], sem0)

    def step(j, carry):
        nxt = j + 1
        even = lax.rem(j, 2) == 0

        @pl.when(jnp.logical_and(nxt < NCHUNKS, even))
        def _():
            pltpu.async_copy(x_hbm.at[sidx(nxt)], rows.at[1], sem1)

        @pl.when(jnp.logical_and(nxt < NCHUNKS, jnp.logical_not(even)))
        def _():
            pltpu.async_copy(x_hbm.at[sidx(nxt)], rows.at[0], sem0)

        @pl.when(even)
        def _():
            pltpu.make_async_copy(x_hbm.at[sidx(j)], rows.at[0], sem0).wait()
            pltpu.sync_copy(rows.at[0], acc.at[dsti.at[j]], add=True)

        @pl.when(jnp.logical_not(even))
        def _():
            pltpu.make_async_copy(x_hbm.at[sidx(j)], rows.at[1], sem1).wait()
            pltpu.sync_copy(rows.at[1], acc.at[dsti.at[j]], add=True)

        return carry

    lax.fori_loop(0, NCHUNKS, step, 0)
    plsc.subcore_barrier()
    row0 = pl.multiple_of(c * N_NODES + s * RPT, 8)
    pltpu.sync_copy(acc.at[pl.ds(s * RPT, RPT)], out_hbm.at[pl.ds(row0, RPT)])

    @pl.when(s == 0)
    def _():
        r0 = pl.multiple_of(c * N_NODES + NS * RPT, 8)
        pltpu.sync_copy(acc.at[pl.ds(NS * RPT, RTAIL)],
                        out_hbm.at[pl.ds(r0, RTAIL)])


BLK = 1000
GRID = N_NODES // BLK


def _dense_relu_body(p0, p1, x, wrel, wroot, b, out):
    acc = jnp.dot(p0[...] + p1[...], wrel[...],
                  preferred_element_type=jnp.float32)
    acc = acc + jnp.dot(x[...], wroot[...],
                        preferred_element_type=jnp.float32)
    acc = acc + b[...]
    out[...] = jnp.maximum(acc, 0.0)


_dense_relu = pl.pallas_call(
    _dense_relu_body,
    grid=(GRID,),
    in_specs=[
        pl.BlockSpec((BLK, D), lambda i: (i, 0)),
        pl.BlockSpec((BLK, D), lambda i: (i, 0)),
        pl.BlockSpec((BLK, D), lambda i: (i, 0)),
        pl.BlockSpec((D, D), lambda i: (0, 0)),
        pl.BlockSpec((D, D), lambda i: (0, 0)),
        pl.BlockSpec((1, D), lambda i: (0, 0)),
    ],
    out_specs=pl.BlockSpec((BLK, D), lambda i: (i, 0)),
    out_shape=jax.ShapeDtypeStruct((N_NODES, D), jnp.float32),
)


def _dense_pool_body(q0, q1, h, wrel, wroot, b, bat, pooled):
    i = pl.program_id(0)
    out = jnp.dot(q0[...] + q1[...], wrel[...],
                  preferred_element_type=jnp.float32)
    out = out + jnp.dot(h[...], wroot[...],
                        preferred_element_type=jnp.float32)
    out = out + b[...]

    @pl.when(i == 0)
    def _():
        pooled[...] = jnp.full((N_POOL, D), -jnp.inf, jnp.float32)

    bt = bat[...]
    rows = [jnp.max(jnp.where(bt == g, out, -jnp.inf), axis=0)
            for g in range(N_POOL)]
    pooled[...] = jnp.maximum(pooled[...], jnp.stack(rows, axis=0))


_dense_pool = pl.pallas_call(
    _dense_pool_body,
    grid=(GRID,),
    in_specs=[
        pl.BlockSpec((BLK, D), lambda i: (i, 0)),
        pl.BlockSpec((BLK, D), lambda i: (i, 0)),
        pl.BlockSpec((BLK, D), lambda i: (i, 0)),
        pl.BlockSpec((D, D), lambda i: (0, 0)),
        pl.BlockSpec((D, D), lambda i: (0, 0)),
        pl.BlockSpec((1, D), lambda i: (0, 0)),
        pl.BlockSpec((BLK, 1), lambda i: (i, 0)),
    ],
    out_specs=pl.BlockSpec((N_POOL, D), lambda i: (0, 0)),
    out_shape=jax.ShapeDtypeStruct((N_POOL, D), jnp.float32),
)


def kernel(x, edge_index, batch, W_rel1, W_root1, b1, W_rel2, W_root2, b2):
    x = x.astype(jnp.float32)
    src = edge_index[0].astype(jnp.int32)
    dst = edge_index[1].astype(jnp.int32).reshape(NW, NCHUNKS, CHUNK)
    bat = batch.astype(jnp.int32).reshape(N_NODES, 1)
    zrows = jnp.zeros((RPT, D), jnp.float32)
    b1r = b1.reshape(1, D)
    b2r = b2.reshape(1, D)

    agg1 = _sc_aggregate(x, src, dst, zrows)
    h = _dense_relu(agg1[:N_NODES], agg1[N_NODES:], x, W_rel1, W_root1, b1r)
    agg2 = _sc_aggregate(h, src, dst, zrows)
    pooled = _dense_pool(agg2[:N_NODES], agg2[N_NODES:], h,
                         W_rel2, W_root2, b2r, bat)
    return pooled


# speedup vs baseline: 11.3298x; 1.1242x over previous
import functools

import jax
import jax.numpy as jnp
from jax import lax
from jax.experimental import pallas as pl
from jax.experimental.pallas import tpu as pltpu
from jax.experimental.pallas import tpu_sc as plsc

N_NODES = 10000
N_EDGES = 320000
D = 128
N_POOL = 8

NC = 2
NS = 16
NW = NC * NS
EPT = N_EDGES // NW
CHUNK = 80
NCHUNKS = EPT // CHUNK
RPT = 624
RTAIL = N_NODES - NS * RPT

_mesh = plsc.VectorSubcoreMesh(
    core_axis_name="c", subcore_axis_name="s", num_cores=NC, num_subcores=NS
)


@functools.partial(
    pl.kernel,
    out_type=jax.ShapeDtypeStruct((NC * N_NODES, D), jnp.float32),
    mesh=_mesh,
    scratch_types=[
        pltpu.VMEM((3, CHUNK), jnp.int32),
        pltpu.VMEM((NCHUNKS, CHUNK), jnp.int32),
        pltpu.VMEM((3, CHUNK, D), jnp.float32),
        pltpu.VMEM_SHARED((N_NODES, D), jnp.float32),
        [pltpu.SemaphoreType.DMA] * 3,
        [pltpu.SemaphoreType.DMA] * 3,
        [pltpu.SemaphoreType.DMA] * 3,
    ],
)
def _sc_aggregate(x_hbm, src_hbm, dst_hbm, z_hbm, out_hbm,
                  sbuf, dsti, rows, acc, isems, gsems, ssems):
    c = lax.axis_index("c")
    s = lax.axis_index("s")
    wid = c * NS + s
    ebase = pl.multiple_of(wid * EPT, 8)
    pltpu.sync_copy(dst_hbm.at[wid], dsti)
    pltpu.sync_copy(z_hbm.at[pl.ds(0, RPT)], acc.at[pl.ds(s * RPT, RPT)])

    @pl.when(s == 0)
    def _():
        pltpu.sync_copy(z_hbm.at[pl.ds(0, RTAIL)],
                        acc.at[pl.ds(NS * RPT, RTAIL)])

    def iload(j, k):
        pltpu.async_copy(
            src_hbm.at[pl.ds(pl.multiple_of(ebase + j * CHUNK, 8), CHUNK)],
            sbuf.at[k], isems[k])

    def iwait(k):
        pltpu.make_async_copy(
            src_hbm.at[pl.ds(0, CHUNK)], sbuf.at[k], isems[k]).wait()

    def gwait(k):
        pltpu.make_async_copy(
            x_hbm.at[sbuf.at[k]], rows.at[k], gsems[k]).wait()

    def swait(k):
        pltpu.make_async_copy(
            rows.at[k], acc.at[dsti.at[0]], ssems[k]).wait()

    iload(0, 0)
    iload(1, 1)
    plsc.subcore_barrier()
    iwait(0)
    pltpu.async_copy(x_hbm.at[sbuf.at[0]], rows.at[0], gsems[0])

    def step(j, carry):
        @pl.when(j + 2 < NCHUNKS)
        def _():
            for k in range(3):
                @pl.when(lax.rem(j + 2, 3) == k)
                def _(k=k):
                    iload(j + 2, k)

        @pl.when(j + 1 < NCHUNKS)
        def _():
            for k in range(3):
                @pl.when(lax.rem(j + 1, 3) == k)
                def _(k=k):
                    @pl.when(j >= 2)
                    def _():
                        swait(k)
                    iwait(k)
                    pltpu.async_copy(
                        x_hbm.at[sbuf.at[k]], rows.at[k], gsems[k])

        for k in range(3):
            @pl.when(lax.rem(j, 3) == k)
            def _(k=k):
                gwait(k)
                pltpu.async_copy(
                    rows.at[k], acc.at[dsti.at[j]], ssems[k], add=True)

        return carry

    lax.fori_loop(0, NCHUNKS, step, 0)
    swait((NCHUNKS - 2) % 3)
    swait((NCHUNKS - 1) % 3)
    plsc.subcore_barrier()
    row0 = pl.multiple_of(c * N_NODES + s * RPT, 8)
    pltpu.sync_copy(acc.at[pl.ds(s * RPT, RPT)], out_hbm.at[pl.ds(row0, RPT)])

    @pl.when(s == 0)
    def _():
        r0 = pl.multiple_of(c * N_NODES + NS * RPT, 8)
        pltpu.sync_copy(acc.at[pl.ds(NS * RPT, RTAIL)],
                        out_hbm.at[pl.ds(r0, RTAIL)])


BLK = 1000
GRID = N_NODES // BLK


def _dense_relu_body(p0, p1, x, wrel, wroot, b, out):
    acc = jnp.dot(p0[...] + p1[...], wrel[...],
                  preferred_element_type=jnp.float32)
    acc = acc + jnp.dot(x[...], wroot[...],
                        preferred_element_type=jnp.float32)
    acc = acc + b[...]
    out[...] = jnp.maximum(acc, 0.0)


_dense_relu = pl.pallas_call(
    _dense_relu_body,
    grid=(GRID,),
    in_specs=[
        pl.BlockSpec((BLK, D), lambda i: (i, 0)),
        pl.BlockSpec((BLK, D), lambda i: (i, 0)),
        pl.BlockSpec((BLK, D), lambda i: (i, 0)),
        pl.BlockSpec((D, D), lambda i: (0, 0)),
        pl.BlockSpec((D, D), lambda i: (0, 0)),
        pl.BlockSpec((1, D), lambda i: (0, 0)),
    ],
    out_specs=pl.BlockSpec((BLK, D), lambda i: (i, 0)),
    out_shape=jax.ShapeDtypeStruct((N_NODES, D), jnp.float32),
)


def _dense_pool_body(q0, q1, h, wrel, wroot, b, bat, pooled):
    i = pl.program_id(0)
    out = jnp.dot(q0[...] + q1[...], wrel[...],
                  preferred_element_type=jnp.float32)
    out = out + jnp.dot(h[...], wroot[...],
                        preferred_element_type=jnp.float32)
    out = out + b[...]

    @pl.when(i == 0)
    def _():
        pooled[...] = jnp.full((N_POOL, D), -jnp.inf, jnp.float32)

    bt = bat[...]
    rows = [jnp.max(jnp.where(bt == g, out, -jnp.inf), axis=0)
            for g in range(N_POOL)]
    pooled[...] = jnp.maximum(pooled[...], jnp.stack(rows, axis=0))


_dense_pool = pl.pallas_call(
    _dense_pool_body,
    grid=(GRID,),
    in_specs=[
        pl.BlockSpec((BLK, D), lambda i: (i, 0)),
        pl.BlockSpec((BLK, D), lambda i: (i, 0)),
        pl.BlockSpec((BLK, D), lambda i: (i, 0)),
        pl.BlockSpec((D, D), lambda i: (0, 0)),
        pl.BlockSpec((D, D), lambda i: (0, 0)),
        pl.BlockSpec((1, D), lambda i: (0, 0)),
        pl.BlockSpec((BLK, 1), lambda i: (i, 0)),
    ],
    out_specs=pl.BlockSpec((N_POOL, D), lambda i: (0, 0)),
    out_shape=jax.ShapeDtypeStruct((N_POOL, D), jnp.float32),
)


def kernel(x, edge_index, batch, W_rel1, W_root1, b1, W_rel2, W_root2, b2):
    x = x.astype(jnp.float32)
    src = edge_index[0].astype(jnp.int32)
    dst = edge_index[1].astype(jnp.int32).reshape(NW, NCHUNKS, CHUNK)
    bat = batch.astype(jnp.int32).reshape(N_NODES, 1)
    zrows = jnp.zeros((RPT, D), jnp.float32)
    b1r = b1.reshape(1, D)
    b2r = b2.reshape(1, D)

    agg1 = _sc_aggregate(x, src, dst, zrows)
    h = _dense_relu(agg1[:N_NODES], agg1[N_NODES:], x, W_rel1, W_root1, b1r)
    agg2 = _sc_aggregate(h, src, dst, zrows)
    pooled = _dense_pool(agg2[:N_NODES], agg2[N_NODES:], h,
                         W_rel2, W_root2, b2r, bat)
    return pooled


# speedup vs baseline: 11.4992x; 1.0150x over previous
import functools

import jax
import jax.numpy as jnp
from jax import lax
from jax.experimental import pallas as pl
from jax.experimental.pallas import tpu as pltpu
from jax.experimental.pallas import tpu_sc as plsc

N_NODES = 10000
N_EDGES = 320000
D = 128
N_POOL = 8

NC = 2
NS = 16
NW = NC * NS
EPT = N_EDGES // NW
CHUNK = 80
NCHUNKS = EPT // CHUNK
RPT = 624
RTAIL = N_NODES - NS * RPT

_mesh = plsc.VectorSubcoreMesh(
    core_axis_name="c", subcore_axis_name="s", num_cores=NC, num_subcores=NS
)


@functools.partial(
    pl.kernel,
    out_type=jax.ShapeDtypeStruct((NC * N_NODES, D), jnp.float32),
    mesh=_mesh,
    scratch_types=[
        pltpu.VMEM((3, CHUNK), jnp.int32),
        pltpu.VMEM((NCHUNKS, CHUNK), jnp.int32),
        pltpu.VMEM((3, CHUNK, D), jnp.float32),
        pltpu.VMEM_SHARED((N_NODES, D), jnp.float32),
        [pltpu.SemaphoreType.DMA] * 3,
        [pltpu.SemaphoreType.DMA] * 3,
        [pltpu.SemaphoreType.DMA] * 3,
    ],
)
def _sc_aggregate(x_hbm, src_hbm, dst_hbm, z_hbm, out_hbm,
                  sbuf, dsti, rows, acc, isems, gsems, ssems):
    c = lax.axis_index("c")
    s = lax.axis_index("s")
    wid = c * NS + s
    ebase = pl.multiple_of(wid * EPT, 8)
    pltpu.sync_copy(dst_hbm.at[wid], dsti)
    pltpu.sync_copy(z_hbm.at[pl.ds(0, RPT)], acc.at[pl.ds(s * RPT, RPT)])

    @pl.when(s == 0)
    def _():
        pltpu.sync_copy(z_hbm.at[pl.ds(0, RTAIL)],
                        acc.at[pl.ds(NS * RPT, RTAIL)])

    def iload(j, k):
        pltpu.async_copy(
            src_hbm.at[pl.ds(pl.multiple_of(ebase + j * CHUNK, 8), CHUNK)],
            sbuf.at[k], isems[k])

    def iwait(k):
        pltpu.make_async_copy(
            src_hbm.at[pl.ds(0, CHUNK)], sbuf.at[k], isems[k]).wait()

    def gwait(k):
        pltpu.make_async_copy(
            x_hbm.at[sbuf.at[k]], rows.at[k], gsems[k]).wait()

    def swait(k):
        pltpu.make_async_copy(
            rows.at[k], acc.at[dsti.at[0]], ssems[k]).wait()

    iload(0, 0)
    iload(1, 1)
    plsc.subcore_barrier()
    iwait(0)
    pltpu.async_copy(x_hbm.at[sbuf.at[0]], rows.at[0], gsems[0])

    def step(j, carry):
        @pl.when(j + 2 < NCHUNKS)
        def _():
            for k in range(3):
                @pl.when(lax.rem(j + 2, 3) == k)
                def _(k=k):
                    iload(j + 2, k)

        @pl.when(j + 1 < NCHUNKS)
        def _():
            for k in range(3):
                @pl.when(lax.rem(j + 1, 3) == k)
                def _(k=k):
                    @pl.when(j >= 2)
                    def _():
                        swait(k)
                    iwait(k)
                    pltpu.async_copy(
                        x_hbm.at[sbuf.at[k]], rows.at[k], gsems[k])

        for k in range(3):
            @pl.when(lax.rem(j, 3) == k)
            def _(k=k):
                gwait(k)
                pltpu.async_copy(
                    rows.at[k], acc.at[dsti.at[j]], ssems[k], add=True)

        return carry

    lax.fori_loop(0, NCHUNKS, step, 0)
    swait((NCHUNKS - 2) % 3)
    swait((NCHUNKS - 1) % 3)
    plsc.subcore_barrier()
    row0 = pl.multiple_of(c * N_NODES + s * RPT, 8)
    pltpu.sync_copy(acc.at[pl.ds(s * RPT, RPT)], out_hbm.at[pl.ds(row0, RPT)])

    @pl.when(s == 0)
    def _():
        r0 = pl.multiple_of(c * N_NODES + NS * RPT, 8)
        pltpu.sync_copy(acc.at[pl.ds(NS * RPT, RTAIL)],
                        out_hbm.at[pl.ds(r0, RTAIL)])


BLK = N_NODES
GRID = N_NODES // BLK


def _dense_relu_body(p0, p1, x, wrel, wroot, b, out):
    acc = jnp.dot(p0[...] + p1[...], wrel[...],
                  preferred_element_type=jnp.float32)
    acc = acc + jnp.dot(x[...], wroot[...],
                        preferred_element_type=jnp.float32)
    acc = acc + b[...]
    out[...] = jnp.maximum(acc, 0.0)


_dense_relu = pl.pallas_call(
    _dense_relu_body,
    grid=(GRID,),
    in_specs=[
        pl.BlockSpec((BLK, D), lambda i: (i, 0)),
        pl.BlockSpec((BLK, D), lambda i: (i, 0)),
        pl.BlockSpec((BLK, D), lambda i: (i, 0)),
        pl.BlockSpec((D, D), lambda i: (0, 0)),
        pl.BlockSpec((D, D), lambda i: (0, 0)),
        pl.BlockSpec((1, D), lambda i: (0, 0)),
    ],
    out_specs=pl.BlockSpec((BLK, D), lambda i: (i, 0)),
    out_shape=jax.ShapeDtypeStruct((N_NODES, D), jnp.float32),
)


def _dense_pool_body(q0, q1, h, wrel, wroot, b, bat, pooled):
    i = pl.program_id(0)
    out = jnp.dot(q0[...] + q1[...], wrel[...],
                  preferred_element_type=jnp.float32)
    out = out + jnp.dot(h[...], wroot[...],
                        preferred_element_type=jnp.float32)
    out = out + b[...]

    @pl.when(i == 0)
    def _():
        pooled[...] = jnp.full((N_POOL, D), -jnp.inf, jnp.float32)

    bt = bat[...]
    rows = [jnp.max(jnp.where(bt == g, out, -jnp.inf), axis=0)
            for g in range(N_POOL)]
    pooled[...] = jnp.maximum(pooled[...], jnp.stack(rows, axis=0))


_dense_pool = pl.pallas_call(
    _dense_pool_body,
    grid=(GRID,),
    in_specs=[
        pl.BlockSpec((BLK, D), lambda i: (i, 0)),
        pl.BlockSpec((BLK, D), lambda i: (i, 0)),
        pl.BlockSpec((BLK, D), lambda i: (i, 0)),
        pl.BlockSpec((D, D), lambda i: (0, 0)),
        pl.BlockSpec((D, D), lambda i: (0, 0)),
        pl.BlockSpec((1, D), lambda i: (0, 0)),
        pl.BlockSpec((BLK, 1), lambda i: (i, 0)),
    ],
    out_specs=pl.BlockSpec((N_POOL, D), lambda i: (0, 0)),
    out_shape=jax.ShapeDtypeStruct((N_POOL, D), jnp.float32),
)


def kernel(x, edge_index, batch, W_rel1, W_root1, b1, W_rel2, W_root2, b2):
    x = x.astype(jnp.float32)
    src = edge_index[0].astype(jnp.int32)
    dst = edge_index[1].astype(jnp.int32).reshape(NW, NCHUNKS, CHUNK)
    bat = batch.astype(jnp.int32).reshape(N_NODES, 1)
    zrows = jnp.zeros((RPT, D), jnp.float32)
    b1r = b1.reshape(1, D)
    b2r = b2.reshape(1, D)

    agg1 = _sc_aggregate(x, src, dst, zrows)
    h = _dense_relu(agg1[:N_NODES], agg1[N_NODES:], x, W_rel1, W_root1, b1r)
    agg2 = _sc_aggregate(h, src, dst, zrows)
    pooled = _dense_pool(agg2[:N_NODES], agg2[N_NODES:], h,
                         W_rel2, W_root2, b2r, bat)
    return pooled


# speedup vs baseline: 12.1976x; 1.0607x over previous
import functools

import jax
import jax.numpy as jnp
from jax import lax
from jax.experimental import pallas as pl
from jax.experimental.pallas import tpu as pltpu
from jax.experimental.pallas import tpu_sc as plsc

N_NODES = 10000
N_EDGES = 320000
D = 128
N_POOL = 8

NC = 2
NS = 16
NW = NC * NS
EPT = N_EDGES // NW
CHUNK = 80
NCHUNKS = EPT // CHUNK
RPT = 624
RTAIL = N_NODES - NS * RPT

_mesh = plsc.VectorSubcoreMesh(
    core_axis_name="c", subcore_axis_name="s", num_cores=NC, num_subcores=NS
)


@functools.partial(
    pl.kernel,
    out_type=jax.ShapeDtypeStruct((NC * N_NODES, D), jnp.float32),
    mesh=_mesh,
    scratch_types=[
        pltpu.VMEM((3, CHUNK), jnp.int32),
        pltpu.VMEM((NCHUNKS, CHUNK), jnp.int32),
        pltpu.VMEM((3, CHUNK, D), jnp.float32),
        pltpu.VMEM_SHARED((N_NODES, D), jnp.float32),
        [pltpu.SemaphoreType.DMA] * 3,
        [pltpu.SemaphoreType.DMA] * 3,
        [pltpu.SemaphoreType.DMA] * 3,
    ],
)
def _sc_aggregate(x_hbm, src_hbm, dst_hbm, z_hbm, out_hbm,
                  sbuf, dsti, rows, acc, isems, gsems, ssems):
    c = lax.axis_index("c")
    s = lax.axis_index("s")
    wid = c * NS + s
    ebase = pl.multiple_of(wid * EPT, 8)
    pltpu.sync_copy(dst_hbm.at[wid], dsti)
    pltpu.sync_copy(z_hbm.at[pl.ds(0, RPT)], acc.at[pl.ds(s * RPT, RPT)])

    @pl.when(s == 0)
    def _():
        pltpu.sync_copy(z_hbm.at[pl.ds(0, RTAIL)],
                        acc.at[pl.ds(NS * RPT, RTAIL)])

    def iload(j, k):
        pltpu.async_copy(
            src_hbm.at[pl.ds(pl.multiple_of(ebase + j * CHUNK, 8), CHUNK)],
            sbuf.at[k], isems[k])

    def iwait(k):
        pltpu.make_async_copy(
            src_hbm.at[pl.ds(0, CHUNK)], sbuf.at[k], isems[k]).wait()

    def gwait(k):
        pltpu.make_async_copy(
            x_hbm.at[sbuf.at[k]], rows.at[k], gsems[k]).wait()

    def swait(k):
        pltpu.make_async_copy(
            rows.at[k], acc.at[dsti.at[0]], ssems[k]).wait()

    iload(0, 0)
    iload(1, 1)
    plsc.subcore_barrier()
    iwait(0)
    pltpu.async_copy(x_hbm.at[sbuf.at[0]], rows.at[0], gsems[0])

    def step(j, carry):
        @pl.when(j + 2 < NCHUNKS)
        def _():
            for k in range(3):
                @pl.when(lax.rem(j + 2, 3) == k)
                def _(k=k):
                    iload(j + 2, k)

        @pl.when(j + 1 < NCHUNKS)
        def _():
            for k in range(3):
                @pl.when(lax.rem(j + 1, 3) == k)
                def _(k=k):
                    @pl.when(j >= 2)
                    def _():
                        swait(k)
                    iwait(k)
                    pltpu.async_copy(
                        x_hbm.at[sbuf.at[k]], rows.at[k], gsems[k])

        for k in range(3):
            @pl.when(lax.rem(j, 3) == k)
            def _(k=k):
                gwait(k)
                pltpu.async_copy(
                    rows.at[k], acc.at[dsti.at[j]], ssems[k], add=True)

        return carry

    lax.fori_loop(0, NCHUNKS, step, 0)
    swait((NCHUNKS - 2) % 3)
    swait((NCHUNKS - 1) % 3)
    plsc.subcore_barrier()
    row0 = pl.multiple_of(c * N_NODES + s * RPT, 8)
    pltpu.sync_copy(acc.at[pl.ds(s * RPT, RPT)], out_hbm.at[pl.ds(row0, RPT)])

    @pl.when(s == 0)
    def _():
        r0 = pl.multiple_of(c * N_NODES + NS * RPT, 8)
        pltpu.sync_copy(acc.at[pl.ds(NS * RPT, RTAIL)],
                        out_hbm.at[pl.ds(r0, RTAIL)])


BLK = N_NODES
GRID = N_NODES // BLK


def _dense_relu_body(pp, x, wrel, wroot, b, out):
    p = pp[...]
    acc = jnp.dot(p[:N_NODES] + p[N_NODES:], wrel[...],
                  preferred_element_type=jnp.float32)
    acc = acc + jnp.dot(x[...], wroot[...],
                        preferred_element_type=jnp.float32)
    acc = acc + b[...]
    out[...] = jnp.maximum(acc, 0.0)


_dense_relu = pl.pallas_call(
    _dense_relu_body,
    grid=(GRID,),
    in_specs=[
        pl.BlockSpec((NC * N_NODES, D), lambda i: (0, 0)),
        pl.BlockSpec((BLK, D), lambda i: (i, 0)),
        pl.BlockSpec((D, D), lambda i: (0, 0)),
        pl.BlockSpec((D, D), lambda i: (0, 0)),
        pl.BlockSpec((1, D), lambda i: (0, 0)),
    ],
    out_specs=pl.BlockSpec((BLK, D), lambda i: (i, 0)),
    out_shape=jax.ShapeDtypeStruct((N_NODES, D), jnp.float32),
)


def _dense_pool_body(qq, h, wrel, wroot, b, bat, pooled):
    i = pl.program_id(0)
    q = qq[...]
    out = jnp.dot(q[:N_NODES] + q[N_NODES:], wrel[...],
                  preferred_element_type=jnp.float32)
    out = out + jnp.dot(h[...], wroot[...],
                        preferred_element_type=jnp.float32)
    out = out + b[...]

    @pl.when(i == 0)
    def _():
        pooled[...] = jnp.full((N_POOL, D), -jnp.inf, jnp.float32)

    bt = bat[...]
    rows = [jnp.max(jnp.where(bt == g, out, -jnp.inf), axis=0)
            for g in range(N_POOL)]
    pooled[...] = jnp.maximum(pooled[...], jnp.stack(rows, axis=0))


_dense_pool = pl.pallas_call(
    _dense_pool_body,
    grid=(GRID,),
    in_specs=[
        pl.BlockSpec((NC * N_NODES, D), lambda i: (0, 0)),
        pl.BlockSpec((BLK, D), lambda i: (i, 0)),
        pl.BlockSpec((D, D), lambda i: (0, 0)),
        pl.BlockSpec((D, D), lambda i: (0, 0)),
        pl.BlockSpec((1, D), lambda i: (0, 0)),
        pl.BlockSpec((BLK, 1), lambda i: (i, 0)),
    ],
    out_specs=pl.BlockSpec((N_POOL, D), lambda i: (0, 0)),
    out_shape=jax.ShapeDtypeStruct((N_POOL, D), jnp.float32),
)


def kernel(x, edge_index, batch, W_rel1, W_root1, b1, W_rel2, W_root2, b2):
    x = x.astype(jnp.float32)
    src = edge_index[0].astype(jnp.int32)
    dst = edge_index[1].astype(jnp.int32).reshape(NW, NCHUNKS, CHUNK)
    bat = batch.astype(jnp.int32).reshape(N_NODES, 1)
    zrows = jnp.zeros((RPT, D), jnp.float32)
    b1r = b1.reshape(1, D)
    b2r = b2.reshape(1, D)

    agg1 = _sc_aggregate(x, src, dst, zrows)
    h = _dense_relu(agg1, x, W_rel1, W_root1, b1r)
    agg2 = _sc_aggregate(h, src, dst, zrows)
    pooled = _dense_pool(agg2, h, W_rel2, W_root2, b2r, bat)
    return pooled


# speedup vs baseline: 12.3634x; 1.0136x over previous
import functools

import jax
import jax.numpy as jnp
from jax import lax
from jax.experimental import pallas as pl
from jax.experimental.pallas import tpu as pltpu
from jax.experimental.pallas import tpu_sc as plsc

N_NODES = 10000
N_EDGES = 320000
D = 128
N_POOL = 8

NC = 2
NS = 16
NW = NC * NS
EPT = N_EDGES // NW
CHUNK = 80
NCHUNKS = EPT // CHUNK
RPT = 624
RTAIL = N_NODES - NS * RPT

_mesh = plsc.VectorSubcoreMesh(
    core_axis_name="c", subcore_axis_name="s", num_cores=NC, num_subcores=NS
)


@functools.partial(
    pl.kernel,
    out_type=jax.ShapeDtypeStruct((NC * N_NODES, D), jnp.float32),
    mesh=_mesh,
    scratch_types=[
        pltpu.VMEM((3, CHUNK), jnp.int32),
        pltpu.VMEM((NCHUNKS, CHUNK), jnp.int32),
        pltpu.VMEM((3, CHUNK, D), jnp.float32),
        pltpu.VMEM_SHARED((N_NODES, D), jnp.float32),
        [pltpu.SemaphoreType.DMA] * 3,
        [pltpu.SemaphoreType.DMA] * 3,
        [pltpu.SemaphoreType.DMA] * 3,
        pltpu.SemaphoreType.DMA,
    ],
)
def _sc_aggregate(x_hbm, src_hbm, dst_hbm, z_hbm, out_hbm,
                  sbuf, dsti, rows, acc, isems, gsems, ssems, zsem):
    c = lax.axis_index("c")
    s = lax.axis_index("s")
    wid = c * NS + s
    ebase = pl.multiple_of(wid * EPT, 8)
    pltpu.async_copy(z_hbm.at[pl.ds(0, RPT)], acc.at[pl.ds(s * RPT, RPT)],
                     zsem)

    @pl.when(s == 0)
    def _():
        pltpu.async_copy(z_hbm.at[pl.ds(0, RTAIL)],
                         acc.at[pl.ds(NS * RPT, RTAIL)], zsem)

    pltpu.sync_copy(dst_hbm.at[wid], dsti)

    def iload(j, k):
        pltpu.async_copy(
            src_hbm.at[pl.ds(pl.multiple_of(ebase + j * CHUNK, 8), CHUNK)],
            sbuf.at[k], isems[k])

    def iwait(k):
        pltpu.make_async_copy(
            src_hbm.at[pl.ds(0, CHUNK)], sbuf.at[k], isems[k]).wait()

    def gwait(k):
        pltpu.make_async_copy(
            x_hbm.at[sbuf.at[k]], rows.at[k], gsems[k]).wait()

    def swait(k):
        pltpu.make_async_copy(
            rows.at[k], acc.at[dsti.at[0]], ssems[k]).wait()

    iload(0, 0)
    iload(1, 1)
    iwait(0)
    pltpu.async_copy(x_hbm.at[sbuf.at[0]], rows.at[0], gsems[0])

    pltpu.make_async_copy(
        z_hbm.at[pl.ds(0, RPT)], acc.at[pl.ds(s * RPT, RPT)], zsem).wait()

    @pl.when(s == 0)
    def _():
        pltpu.make_async_copy(
            z_hbm.at[pl.ds(0, RTAIL)], acc.at[pl.ds(NS * RPT, RTAIL)],
            zsem).wait()

    plsc.subcore_barrier()

    def step(j, carry):
        @pl.when(j + 2 < NCHUNKS)
        def _():
            for k in range(3):
                @pl.when(lax.rem(j + 2, 3) == k)
                def _(k=k):
                    iload(j + 2, k)

        @pl.when(j + 1 < NCHUNKS)
        def _():
            for k in range(3):
                @pl.when(lax.rem(j + 1, 3) == k)
                def _(k=k):
                    @pl.when(j >= 2)
                    def _():
                        swait(k)
                    iwait(k)
                    pltpu.async_copy(
                        x_hbm.at[sbuf.at[k]], rows.at[k], gsems[k])

        for k in range(3):
            @pl.when(lax.rem(j, 3) == k)
            def _(k=k):
                gwait(k)
                pltpu.async_copy(
                    rows.at[k], acc.at[dsti.at[j]], ssems[k], add=True)

        return carry

    lax.fori_loop(0, NCHUNKS, step, 0)
    swait((NCHUNKS - 2) % 3)
    swait((NCHUNKS - 1) % 3)
    plsc.subcore_barrier()
    row0 = pl.multiple_of(c * N_NODES + s * RPT, 8)
    pltpu.sync_copy(acc.at[pl.ds(s * RPT, RPT)], out_hbm.at[pl.ds(row0, RPT)])

    @pl.when(s == 0)
    def _():
        r0 = pl.multiple_of(c * N_NODES + NS * RPT, 8)
        pltpu.sync_copy(acc.at[pl.ds(NS * RPT, RTAIL)],
                        out_hbm.at[pl.ds(r0, RTAIL)])


BLK = N_NODES
GRID = N_NODES // BLK


def _dense_relu_body(pp, x, wrel, wroot, b, out):
    p = pp[...]
    acc = jnp.dot(p[:N_NODES] + p[N_NODES:], wrel[...],
                  preferred_element_type=jnp.float32)
    acc = acc + jnp.dot(x[...], wroot[...],
                        preferred_element_type=jnp.float32)
    acc = acc + b[...]
    out[...] = jnp.maximum(acc, 0.0)


_dense_relu = pl.pallas_call(
    _dense_relu_body,
    grid=(GRID,),
    in_specs=[
        pl.BlockSpec((NC * N_NODES, D), lambda i: (0, 0)),
        pl.BlockSpec((BLK, D), lambda i: (i, 0)),
        pl.BlockSpec((D, D), lambda i: (0, 0)),
        pl.BlockSpec((D, D), lambda i: (0, 0)),
        pl.BlockSpec((1, D), lambda i: (0, 0)),
    ],
    out_specs=pl.BlockSpec((BLK, D), lambda i: (i, 0)),
    out_shape=jax.ShapeDtypeStruct((N_NODES, D), jnp.float32),
)


def _dense_pool_body(qq, h, wrel, wroot, b, bat, pooled):
    i = pl.program_id(0)
    q = qq[...]
    out = jnp.dot(q[:N_NODES] + q[N_NODES:], wrel[...],
                  preferred_element_type=jnp.float32)
    out = out + jnp.dot(h[...], wroot[...],
                        preferred_element_type=jnp.float32)
    out = out + b[...]

    @pl.when(i == 0)
    def _():
        pooled[...] = jnp.full((N_POOL, D), -jnp.inf, jnp.float32)

    bt = bat[...]
    rows = [jnp.max(jnp.where(bt == g, out, -jnp.inf), axis=0)
            for g in range(N_POOL)]
    pooled[...] = jnp.maximum(pooled[...], jnp.stack(rows, axis=0))


_dense_pool = pl.pallas_call(
    _dense_pool_body,
    grid=(GRID,),
    in_specs=[
        pl.BlockSpec((NC * N_NODES, D), lambda i: (0, 0)),
        pl.BlockSpec((BLK, D), lambda i: (i, 0)),
        pl.BlockSpec((D, D), lambda i: (0, 0)),
        pl.BlockSpec((D, D), lambda i: (0, 0)),
        pl.BlockSpec((1, D), lambda i: (0, 0)),
        pl.BlockSpec((BLK, 1), lambda i: (i, 0)),
    ],
    out_specs=pl.BlockSpec((N_POOL, D), lambda i: (0, 0)),
    out_shape=jax.ShapeDtypeStruct((N_POOL, D), jnp.float32),
)


def kernel(x, edge_index, batch, W_rel1, W_root1, b1, W_rel2, W_root2, b2):
    x = x.astype(jnp.float32)
    src = edge_index[0].astype(jnp.int32)
    dst = edge_index[1].astype(jnp.int32).reshape(NW, NCHUNKS, CHUNK)
    bat = batch.astype(jnp.int32).reshape(N_NODES, 1)
    zrows = jnp.zeros((RPT, D), jnp.float32)
    b1r = b1.reshape(1, D)
    b2r = b2.reshape(1, D)

    agg1 = _sc_aggregate(x, src, dst, zrows)
    h = _dense_relu(agg1, x, W_rel1, W_root1, b1r)
    agg2 = _sc_aggregate(h, src, dst, zrows)
    pooled = _dense_pool(agg2, h, W_rel2, W_root2, b2r, bat)
    return pooled
